# baseline (speedup 1.0000x reference)
import functools

import jax
import jax.numpy as jnp
from jax import lax
from jax.experimental import pallas as pl
from jax.experimental.pallas import tpu as pltpu
from jax.experimental.pallas import tpu_sc as plsc

NUM_ANNOTATORS = 100000
LATENT_DIMS = 16
BATCH = 16384

_PACK = 128 // LATENT_DIMS
_SUPER_ROWS = NUM_ANNOTATORS // _PACK
_NUM_CORES = 2
_NUM_SUBCORES = 16
_NW = _NUM_CORES * _NUM_SUBCORES
_B_PER_W = BATCH // _NW
_OUT_ROWS = BATCH * LATENT_DIMS // 128
_OUT_PER_W = _OUT_ROWS // _NW


@functools.partial(
    pl.kernel,
    mesh=plsc.VectorSubcoreMesh(core_axis_name="c", subcore_axis_name="s"),
    out_type=jax.ShapeDtypeStruct((_OUT_ROWS, 128), jnp.float32),
    scratch_types=[
        pltpu.VMEM((_B_PER_W,), jnp.int32),
        pltpu.VMEM((_B_PER_W,), jnp.int32),
        pltpu.VMEM((_B_PER_W,), jnp.int32),
        pltpu.VMEM((_B_PER_W, 128), jnp.float32),
        pltpu.VMEM((_OUT_PER_W, 128), jnp.float32),
        pltpu.SemaphoreType.DMA,
    ],
    compiler_params=pltpu.CompilerParams(needs_layout_passes=False),
)
def _gather_sc(idx_hbm, table_hbm, out_hbm, idx_v, sup_v, cb_v, rows_v,
               out_v, sem):
    wid = lax.axis_index("s") * _NUM_CORES + lax.axis_index("c")
    base = wid * _B_PER_W
    pltpu.sync_copy(idx_hbm.at[pl.ds(base, _B_PER_W)], idx_v)

    for j in range(_B_PER_W // 16):
        v = idx_v[pl.ds(j * 16, 16)]
        sup_v[pl.ds(j * 16, 16)] = lax.shift_right_logical(v, 3)
        cb_v[pl.ds(j * 16, 16)] = lax.shift_left(
            lax.bitwise_and(v, jnp.int32(7)), 4)

    pltpu.async_copy(table_hbm.at[sup_v], rows_v, sem).wait()

    lanes = lax.iota(jnp.int32, 16)

    def body(b, _):
        i0 = b * 16
        cb16 = cb_v[pl.ds(i0, 16)]
        for k in range(16):
            i = i0 + k
            cols = cb16[k] + lanes
            row = jnp.full((16,), i, jnp.int32)
            vals = plsc.load_gather(rows_v, [row, cols])
            orow = jnp.full((16,), lax.shift_right_logical(i, 3), jnp.int32)
            ocols = (k % 8) * 16 + lanes
            plsc.store_scatter(out_v, [orow, ocols], vals)
        return 0

    lax.fori_loop(0, _B_PER_W // 16, body, 0)

    pltpu.sync_copy(out_v, out_hbm.at[pl.ds(wid * _OUT_PER_W, _OUT_PER_W)])


def kernel(annotator, z_vecs):
    table = z_vecs.reshape(_SUPER_ROWS, 128)
    out = _gather_sc(annotator.astype(jnp.int32), table)
    return out.reshape(BATCH, LATENT_DIMS)


# speedup vs baseline: 1.1258x; 1.1258x over previous
import functools

import jax
import jax.numpy as jnp
from jax import lax
from jax.experimental import pallas as pl
from jax.experimental.pallas import tpu as pltpu
from jax.experimental.pallas import tpu_sc as plsc

NUM_ANNOTATORS = 100000
LATENT_DIMS = 16
BATCH = 16384

_PACK = 128 // LATENT_DIMS
_SUPER_ROWS = NUM_ANNOTATORS // _PACK
_NUM_CORES = 2
_NUM_SUBCORES = 16
_NW = _NUM_CORES * _NUM_SUBCORES
_B_PER_W = BATCH // _NW


@functools.partial(
    pl.kernel,
    mesh=plsc.VectorSubcoreMesh(core_axis_name="c", subcore_axis_name="s"),
    out_type=jax.ShapeDtypeStruct((LATENT_DIMS, BATCH), jnp.float32),
    scratch_types=[
        pltpu.VMEM((_B_PER_W,), jnp.int32),
        pltpu.VMEM((_B_PER_W,), jnp.int32),
        pltpu.VMEM((_B_PER_W,), jnp.int32),
        pltpu.VMEM((_B_PER_W, 128), jnp.float32),
        pltpu.VMEM((LATENT_DIMS, _B_PER_W), jnp.float32),
        pltpu.SemaphoreType.DMA,
    ],
    compiler_params=pltpu.CompilerParams(needs_layout_passes=False),
)
def _gather_sc(idx_hbm, table_hbm, outT_hbm, idx_v, sup_v, cb_v, rows_v,
               outT_v, sem):
    wid = lax.axis_index("s") * _NUM_CORES + lax.axis_index("c")
    base = wid * _B_PER_W
    pltpu.sync_copy(idx_hbm.at[pl.ds(base, _B_PER_W)], idx_v)

    for j in range(_B_PER_W // 16):
        v = idx_v[pl.ds(j * 16, 16)]
        sup_v[pl.ds(j * 16, 16)] = lax.shift_right_logical(v, 3)
        cb_v[pl.ds(j * 16, 16)] = lax.shift_left(
            lax.bitwise_and(v, jnp.int32(7)), 4)

    pltpu.async_copy(table_hbm.at[sup_v], rows_v, sem).wait()

    rows16 = lax.iota(jnp.int32, 16)

    def body(c, _):
        i0 = c * 16
        cb16 = cb_v[pl.ds(i0, 16)]
        rloc = i0 + rows16
        for j in range(LATENT_DIMS):
            vals = plsc.load_gather(rows_v, [rloc, cb16 + j])
            outT_v[j, pl.ds(i0, 16)] = vals
        return 0

    lax.fori_loop(0, _B_PER_W // 16, body, 0)

    for j in range(LATENT_DIMS):
        pltpu.sync_copy(outT_v.at[j], outT_hbm.at[j, pl.ds(base, _B_PER_W)])


def kernel(annotator, z_vecs):
    table = z_vecs.reshape(_SUPER_ROWS, 128)
    outT = _gather_sc(annotator.astype(jnp.int32), table)
    return outT.T


# speedup vs baseline: 1.1981x; 1.0642x over previous
import functools

import jax
import jax.numpy as jnp
from jax import lax
from jax.experimental import pallas as pl
from jax.experimental.pallas import tpu as pltpu
from jax.experimental.pallas import tpu_sc as plsc

NUM_ANNOTATORS = 100000
LATENT_DIMS = 16
BATCH = 16384

_NUM_CORES = 2
_NUM_SUBCORES = 16
_NW = _NUM_CORES * _NUM_SUBCORES
_B_PER_W = BATCH // _NW
_NWORDS = _B_PER_W * LATENT_DIMS
_FULL_BLOCKS = NUM_ANNOTATORS // 128
_TAIL_START = _FULL_BLOCKS * 128
_PITCH = _TAIL_START + 128
_BLK_PER_SUB = 98
_NSLOTS = 6


@functools.partial(
    pl.kernel,
    mesh=plsc.VectorSubcoreMesh(core_axis_name="c", subcore_axis_name="s"),
    out_type=jax.ShapeDtypeStruct((LATENT_DIMS, BATCH), jnp.float32),
    scratch_types=[
        pltpu.VMEM((_B_PER_W,), jnp.int32),
        pltpu.VMEM((_NWORDS,), jnp.int32),
        pltpu.VMEM((_NWORDS,), jnp.float32),
        [pltpu.VMEM((8, 128), jnp.float32) for _ in range(_NSLOTS)],
        [pltpu.VMEM((8, 128), jnp.float32) for _ in range(2)],
        pltpu.VMEM_SHARED((LATENT_DIMS * _PITCH,), jnp.float32),
        pltpu.SemaphoreType.DMA,
        pltpu.SemaphoreType.DMA,
        pltpu.SemaphoreType.DMA,
    ],
)
def _gather_sc(idx_hbm, zt3_hbm, tail_hbm, outT_hbm, idx_v, wix_v, words_v,
               slots, tails, table_sh, sem_in, sem_out, sem):
    sid = lax.axis_index("s")
    wid = sid * _NUM_CORES + lax.axis_index("c")
    base = wid * _B_PER_W
    jb = lax.bitwise_and(sid, jnp.int32(1))
    s8 = lax.shift_right_logical(sid, 1)
    flat_jb = jb * (8 * _PITCH)

    def blk(t):
        return lax.min(s8 * _BLK_PER_SUB + t, jnp.int32(_FULL_BLOCKS - 1))

    def fire_in(t, s):
        return pltpu.async_copy(
            zt3_hbm.at[jb, :, pl.ds(blk(t) * 128, 128)], slots[s], sem_in)

    def fire_outs(t, s):
        b128 = blk(t) * 128
        return [
            pltpu.async_copy(
                slots[s].at[r],
                table_sh.at[pl.ds(flat_jb + r * _PITCH + b128, 128)],
                sem_out)
            for r in range(8)
        ]

    ind = [None] * _NSLOTS
    outd = [None] * _NSLOTS
    for t in range(_BLK_PER_SUB + 2):
        if t >= 2:
            u = t - 2
            su = u % _NSLOTS
            ind[su].wait()
            outd[su] = fire_outs(u, su)
        if t < _BLK_PER_SUB:
            s = t % _NSLOTS
            if outd[s] is not None:
                for d in outd[s]:
                    d.wait()
                outd[s] = None
            ind[s] = fire_in(t, s)
        if t == 0:
            pltpu.sync_copy(idx_hbm.at[pl.ds(base, _B_PER_W)], idx_v)

    @pl.when(sid == 0)
    def _():
        for h in range(2):
            pltpu.sync_copy(tail_hbm.at[pl.ds(h * 8, 8), :], tails[h])
            for r in range(8):
                pltpu.sync_copy(
                    tails[h].at[r],
                    table_sh.at[pl.ds((h * 8 + r) * _PITCH + _TAIL_START,
                                      128)])

    def wix(c, _):
        a16 = idx_v[pl.ds(c * 16, 16)]
        for j in range(LATENT_DIMS):
            wix_v[pl.ds(j * _B_PER_W + c * 16, 16)] = a16 + j * _PITCH
        return 0

    lax.fori_loop(0, _B_PER_W // 16, wix, 0)

    for s in range(_NSLOTS):
        if outd[s] is not None:
            for d in outd[s]:
                d.wait()
    plsc.subcore_barrier()

    pltpu.async_copy(table_sh.at[wix_v], words_v, sem).wait()

    for j in range(LATENT_DIMS):
        pltpu.sync_copy(words_v.at[pl.ds(j * _B_PER_W, _B_PER_W)],
                        outT_hbm.at[j, pl.ds(base, _B_PER_W)])


def kernel(annotator, z_vecs):
    zt3 = z_vecs.T.reshape(2, 8, NUM_ANNOTATORS)
    tail = jnp.pad(z_vecs.T[:, _TAIL_START:],
                   ((0, 0), (0, _PITCH - NUM_ANNOTATORS)))
    outT = _gather_sc(annotator.astype(jnp.int32), zt3, tail)
    return outT.T


# speedup vs baseline: 1.7895x; 1.4937x over previous
import functools

import jax
import jax.numpy as jnp
from jax import lax
from jax.experimental import pallas as pl
from jax.experimental.pallas import tpu as pltpu
from jax.experimental.pallas import tpu_sc as plsc

NUM_ANNOTATORS = 100000
LATENT_DIMS = 16
BATCH = 16384

_NUM_CORES = 2
_NUM_SUBCORES = 16
_NW = _NUM_CORES * _NUM_SUBCORES
_B_PER_W = BATCH // _NW
_NWORDS = _B_PER_W * LATENT_DIMS
_FULL_BLOCKS = NUM_ANNOTATORS // 128
_TAIL_START = _FULL_BLOCKS * 128
_PITCH = _TAIL_START + 128
_BLK_PER_SUB = 98
_NSLOTS = 20
_DELAY = 10


@functools.partial(
    pl.kernel,
    mesh=plsc.VectorSubcoreMesh(core_axis_name="c", subcore_axis_name="s"),
    out_type=jax.ShapeDtypeStruct((LATENT_DIMS, BATCH), jnp.float32),
    scratch_types=[
        pltpu.VMEM((_B_PER_W,), jnp.int32),
        pltpu.VMEM((_NWORDS,), jnp.float32),
        [pltpu.VMEM((8, 128), jnp.float32) for _ in range(_NSLOTS)],
        pltpu.VMEM_SHARED((LATENT_DIMS * _PITCH,), jnp.float32),
        pltpu.SemaphoreType.DMA,
        pltpu.SemaphoreType.DMA,
        pltpu.SemaphoreType.DMA,
    ],
)
def _gather_sc(idx_hbm, zt3_hbm, tail_hbm, outT_hbm, idx_v, words_v,
               slots, table_sh, sem_in, sem_out, sem):
    sid = lax.axis_index("s")
    wid = sid * _NUM_CORES + lax.axis_index("c")
    base = wid * _B_PER_W
    jb = lax.bitwise_and(sid, jnp.int32(1))
    s8 = lax.shift_right_logical(sid, 1)
    flat_jb = jb * (8 * _PITCH)

    def blk(t):
        return lax.min(s8 * _BLK_PER_SUB + t, jnp.int32(_FULL_BLOCKS - 1))

    def fire_in(t, s):
        return pltpu.async_copy(
            zt3_hbm.at[jb, :, pl.ds(blk(t) * 128, 128)], slots[s], sem_in)

    def fire_outs(t, s):
        b128 = blk(t) * 128
        return [
            pltpu.async_copy(
                slots[s].at[r],
                table_sh.at[pl.ds(flat_jb + r * _PITCH + b128, 128)],
                sem_out)
            for r in range(8)
        ]

    ind = [None] * _NSLOTS
    outd = [None] * _NSLOTS
    for t in range(_BLK_PER_SUB + _DELAY):
        if t >= _DELAY:
            u = t - _DELAY
            su = u % _NSLOTS
            ind[su].wait()
            outd[su] = fire_outs(u, su)
        if t < _BLK_PER_SUB:
            s = t % _NSLOTS
            if outd[s] is not None:
                for d in outd[s]:
                    d.wait()
                outd[s] = None
            ind[s] = fire_in(t, s)
        if t == 0:
            pltpu.sync_copy(idx_hbm.at[pl.ds(base, _B_PER_W)], idx_v)

    for s in range(_NSLOTS):
        if outd[s] is not None:
            for d in outd[s]:
                d.wait()

    @pl.when(sid == 0)
    def _():
        for h in range(2):
            pltpu.sync_copy(tail_hbm.at[pl.ds(h * 8, 8), :], slots[h])
            for r in range(8):
                pltpu.sync_copy(
                    slots[h].at[r],
                    table_sh.at[pl.ds((h * 8 + r) * _PITCH + _TAIL_START,
                                      128)])

    plsc.subcore_barrier()

    gathers = [
        pltpu.async_copy(
            table_sh.at[pl.ds(j * _PITCH, _PITCH)].at[idx_v],
            words_v.at[pl.ds(j * _B_PER_W, _B_PER_W)],
            sem)
        for j in range(LATENT_DIMS)
    ]
    for g in gathers:
        g.wait()

    for j in range(LATENT_DIMS):
        pltpu.sync_copy(words_v.at[pl.ds(j * _B_PER_W, _B_PER_W)],
                        outT_hbm.at[j, pl.ds(base, _B_PER_W)])


def kernel(annotator, z_vecs):
    zt3 = z_vecs.T.reshape(2, 8, NUM_ANNOTATORS)
    tail = jnp.pad(z_vecs.T[:, _TAIL_START:],
                   ((0, 0), (0, _PITCH - NUM_ANNOTATORS)))
    outT = _gather_sc(annotator.astype(jnp.int32), zt3, tail)
    return outT.T


# speedup vs baseline: 1.8326x; 1.0241x over previous
import functools

import jax
import jax.numpy as jnp
from jax import lax
from jax.experimental import pallas as pl
from jax.experimental.pallas import tpu as pltpu
from jax.experimental.pallas import tpu_sc as plsc

NUM_ANNOTATORS = 100000
LATENT_DIMS = 16
BATCH = 16384

_NUM_CORES = 2
_NUM_SUBCORES = 16
_NW = _NUM_CORES * _NUM_SUBCORES
_B_PER_W = BATCH // _NW
_NWORDS = _B_PER_W * LATENT_DIMS
_FULL_BLOCKS = NUM_ANNOTATORS // 128
_TAIL_START = _FULL_BLOCKS * 128
_PITCH = _TAIL_START + 128
_BLK_PER_SUB = 98
_NSLOTS = 20
_DELAY = 14


@functools.partial(
    pl.kernel,
    mesh=plsc.VectorSubcoreMesh(core_axis_name="c", subcore_axis_name="s"),
    out_type=jax.ShapeDtypeStruct((LATENT_DIMS, BATCH), jnp.float32),
    scratch_types=[
        pltpu.VMEM((_B_PER_W,), jnp.int32),
        pltpu.VMEM((_NWORDS,), jnp.float32),
        [pltpu.VMEM((8, 128), jnp.float32) for _ in range(_NSLOTS)],
        pltpu.VMEM_SHARED((LATENT_DIMS * _PITCH,), jnp.float32),
        pltpu.SemaphoreType.DMA,
        pltpu.SemaphoreType.DMA,
        pltpu.SemaphoreType.DMA,
    ],
)
def _gather_sc(idx_hbm, zt3_hbm, tail_hbm, outT_hbm, idx_v, words_v,
               slots, table_sh, sem_in, sem_out, sem):
    sid = lax.axis_index("s")
    wid = sid * _NUM_CORES + lax.axis_index("c")
    base = wid * _B_PER_W
    jb = lax.bitwise_and(sid, jnp.int32(1))
    s8 = lax.shift_right_logical(sid, 1)
    flat_jb = jb * (8 * _PITCH)

    def blk(t):
        return lax.min(s8 * _BLK_PER_SUB + t, jnp.int32(_FULL_BLOCKS - 1))

    def fire_in(t, s):
        return pltpu.async_copy(
            zt3_hbm.at[jb, :, pl.ds(blk(t) * 128, 128)], slots[s], sem_in)

    def fire_outs(t, s):
        b128 = blk(t) * 128
        return [
            pltpu.async_copy(
                slots[s].at[r],
                table_sh.at[pl.ds(flat_jb + r * _PITCH + b128, 128)],
                sem_out)
            for r in range(8)
        ]

    ind = [None] * _NSLOTS
    outd = [None] * _NSLOTS
    for t in range(_BLK_PER_SUB + _DELAY):
        if t >= _DELAY:
            u = t - _DELAY
            su = u % _NSLOTS
            ind[su].wait()
            outd[su] = fire_outs(u, su)
        if t < _BLK_PER_SUB:
            s = t % _NSLOTS
            if outd[s] is not None:
                for d in outd[s]:
                    d.wait()
                outd[s] = None
            ind[s] = fire_in(t, s)
        if t == 0:
            pltpu.sync_copy(idx_hbm.at[pl.ds(base, _B_PER_W)], idx_v)

    for s in range(_NSLOTS):
        if outd[s] is not None:
            for d in outd[s]:
                d.wait()

    for h in range(2):
        @pl.when(sid == h)
        def _(h=h):
            pltpu.sync_copy(tail_hbm.at[pl.ds(h * 8, 8), :], slots[0])
            tds = [
                pltpu.async_copy(
                    slots[0].at[r],
                    table_sh.at[pl.ds((h * 8 + r) * _PITCH + _TAIL_START,
                                      128)],
                    sem_out)
                for r in range(8)
            ]
            for d in tds:
                d.wait()

    plsc.subcore_barrier()

    gathers = [
        pltpu.async_copy(
            table_sh.at[pl.ds(j * _PITCH, _PITCH)].at[idx_v],
            words_v.at[pl.ds(j * _B_PER_W, _B_PER_W)],
            sem)
        for j in range(LATENT_DIMS)
    ]
    for g in gathers:
        g.wait()

    for j in range(LATENT_DIMS):
        pltpu.sync_copy(words_v.at[pl.ds(j * _B_PER_W, _B_PER_W)],
                        outT_hbm.at[j, pl.ds(base, _B_PER_W)])


def kernel(annotator, z_vecs):
    zt3 = z_vecs.T.reshape(2, 8, NUM_ANNOTATORS)
    tail = jnp.pad(z_vecs.T[:, _TAIL_START:],
                   ((0, 0), (0, _PITCH - NUM_ANNOTATORS)))
    outT = _gather_sc(annotator.astype(jnp.int32), zt3, tail)
    return outT.T


# speedup vs baseline: 1.8646x; 1.0175x over previous
import functools

import jax
import jax.numpy as jnp
from jax import lax
from jax.experimental import pallas as pl
from jax.experimental.pallas import tpu as pltpu
from jax.experimental.pallas import tpu_sc as plsc

NUM_ANNOTATORS = 100000
LATENT_DIMS = 16
BATCH = 16384

_NUM_CORES = 2
_NUM_SUBCORES = 16
_NW = _NUM_CORES * _NUM_SUBCORES
_B_PER_W = BATCH // _NW
_NWORDS = _B_PER_W * LATENT_DIMS
_FULL_BLOCKS = NUM_ANNOTATORS // 128
_TAIL_START = _FULL_BLOCKS * 128
_PITCH = _TAIL_START + 128
_BLK_PER_SUB = 98
_NSLOTS = 25
_DELAY = 18


@functools.partial(
    pl.kernel,
    mesh=plsc.VectorSubcoreMesh(core_axis_name="c", subcore_axis_name="s"),
    out_type=jax.ShapeDtypeStruct((LATENT_DIMS, BATCH), jnp.float32),
    scratch_types=[
        pltpu.VMEM((_B_PER_W,), jnp.int32),
        [pltpu.VMEM((4 * _B_PER_W,), jnp.float32) for _ in range(2)],
        [pltpu.VMEM((8, 128), jnp.float32) for _ in range(_NSLOTS)],
        pltpu.VMEM_SHARED((LATENT_DIMS * _PITCH,), jnp.float32),
        pltpu.SemaphoreType.DMA,
        pltpu.SemaphoreType.DMA,
        pltpu.SemaphoreType.DMA,
    ],
)
def _gather_sc(idx_hbm, zt3_hbm, tail_hbm, outT_hbm, idx_v, words2,
               slots, table_sh, sem_in, sem_out, sem):
    sid = lax.axis_index("s")
    wid = sid * _NUM_CORES + lax.axis_index("c")
    base = wid * _B_PER_W
    jb = lax.bitwise_and(sid, jnp.int32(1))
    s8 = lax.shift_right_logical(sid, 1)
    flat_jb = jb * (8 * _PITCH)

    def blk(t):
        return lax.min(s8 * _BLK_PER_SUB + t, jnp.int32(_FULL_BLOCKS - 1))

    def fire_in(t, s):
        return pltpu.async_copy(
            zt3_hbm.at[jb, :, pl.ds(blk(t) * 128, 128)], slots[s], sem_in)

    def fire_outs(t, s):
        b128 = blk(t) * 128
        return [
            pltpu.async_copy(
                slots[s].at[r],
                table_sh.at[pl.ds(flat_jb + r * _PITCH + b128, 128)],
                sem_out)
            for r in range(8)
        ]

    ind = [None] * _NSLOTS
    outd = [None] * _NSLOTS
    for t in range(_BLK_PER_SUB + _DELAY):
        if t >= _DELAY:
            u = t - _DELAY
            su = u % _NSLOTS
            ind[su].wait()
            outd[su] = fire_outs(u, su)
        if t < _BLK_PER_SUB:
            s = t % _NSLOTS
            if outd[s] is not None:
                for d in outd[s]:
                    d.wait()
                outd[s] = None
            ind[s] = fire_in(t, s)
        if t == 0:
            pltpu.sync_copy(idx_hbm.at[pl.ds(base, _B_PER_W)], idx_v)

    for s in range(_NSLOTS):
        if outd[s] is not None:
            for d in outd[s]:
                d.wait()

    for h in range(2):
        @pl.when(sid == h)
        def _(h=h):
            pltpu.sync_copy(tail_hbm.at[pl.ds(h * 8, 8), :], slots[0])
            tds = [
                pltpu.async_copy(
                    slots[0].at[r],
                    table_sh.at[pl.ds((h * 8 + r) * _PITCH + _TAIL_START,
                                      128)],
                    sem_out)
                for r in range(8)
            ]
            for d in tds:
                d.wait()

    plsc.subcore_barrier()

    outw = [None, None]
    for w in range(4):
        buf = words2[w % 2]
        if outw[w % 2] is not None:
            for d in outw[w % 2]:
                d.wait()
        gs = [
            pltpu.async_copy(
                table_sh.at[pl.ds((4 * w + k) * _PITCH, _PITCH)].at[idx_v],
                buf.at[pl.ds(k * _B_PER_W, _B_PER_W)],
                sem)
            for k in range(4)
        ]
        for g in gs:
            g.wait()
        outw[w % 2] = [
            pltpu.async_copy(
                buf.at[pl.ds(k * _B_PER_W, _B_PER_W)],
                outT_hbm.at[4 * w + k, pl.ds(base, _B_PER_W)],
                sem_out)
            for k in range(4)
        ]
    for ds_ in outw:
        for d in ds_:
            d.wait()


def kernel(annotator, z_vecs):
    zt3 = z_vecs.T.reshape(2, 8, NUM_ANNOTATORS)
    tail = jnp.pad(z_vecs.T[:, _TAIL_START:],
                   ((0, 0), (0, _PITCH - NUM_ANNOTATORS)))
    outT = _gather_sc(annotator.astype(jnp.int32), zt3, tail)
    return outT.T


# speedup vs baseline: 1.8720x; 1.0040x over previous
import functools

import jax
import jax.numpy as jnp
from jax import lax
from jax.experimental import pallas as pl
from jax.experimental.pallas import tpu as pltpu
from jax.experimental.pallas import tpu_sc as plsc

NUM_ANNOTATORS = 100000
LATENT_DIMS = 16
BATCH = 16384

_NUM_CORES = 2
_NUM_SUBCORES = 16
_NW = _NUM_CORES * _NUM_SUBCORES
_B_PER_W = BATCH // _NW
_NWORDS = _B_PER_W * LATENT_DIMS
_FULL_BLOCKS = NUM_ANNOTATORS // 128
_TAIL_START = _FULL_BLOCKS * 128
_PITCH = _TAIL_START + 128
_BLK_PER_SUB = 98
_NSLOTS = 25
_DELAY = 18


@functools.partial(
    pl.kernel,
    mesh=plsc.VectorSubcoreMesh(core_axis_name="c", subcore_axis_name="s"),
    out_type=jax.ShapeDtypeStruct((LATENT_DIMS, BATCH), jnp.float32),
    scratch_types=[
        pltpu.VMEM((_B_PER_W,), jnp.int32),
        [pltpu.VMEM((4 * _B_PER_W,), jnp.float32) for _ in range(2)],
        [pltpu.VMEM((8, 128), jnp.float32) for _ in range(_NSLOTS)],
        pltpu.VMEM_SHARED((LATENT_DIMS * _PITCH,), jnp.float32),
        pltpu.SemaphoreType.DMA,
        pltpu.SemaphoreType.DMA,
        pltpu.SemaphoreType.DMA,
    ],
)
def _gather_sc(idx_hbm, zt3_hbm, tail_hbm, outT_hbm, idx_v, words2,
               slots, table_sh, sem_in, sem_out, sem):
    sid = lax.axis_index("s")
    wid = sid * _NUM_CORES + lax.axis_index("c")
    base = wid * _B_PER_W
    jb = lax.bitwise_and(sid, jnp.int32(1))
    s8 = lax.shift_right_logical(sid, 1)
    flat_jb = jb * (8 * _PITCH)

    def blk(t):
        return lax.min(s8 * _BLK_PER_SUB + t, jnp.int32(_FULL_BLOCKS - 1))

    def fire_in(t, s):
        return pltpu.async_copy(
            zt3_hbm.at[jb, :, pl.ds(blk(t) * 128, 128)], slots[s], sem_in)

    def fire_outs(t, s):
        b128 = blk(t) * 128
        return [
            pltpu.async_copy(
                slots[s].at[r],
                table_sh.at[pl.ds(flat_jb + r * _PITCH + b128, 128)],
                sem_out)
            for r in range(8)
        ]

    ind = [None] * _NSLOTS
    outd = [None] * _NSLOTS
    scope_stage = jax.named_scope("stage")
    scope_stage.__enter__()
    for t in range(_BLK_PER_SUB + _DELAY):
        if t >= _DELAY:
            u = t - _DELAY
            su = u % _NSLOTS
            ind[su].wait()
            outd[su] = fire_outs(u, su)
        if t < _BLK_PER_SUB:
            s = t % _NSLOTS
            if outd[s] is not None:
                for d in outd[s]:
                    d.wait()
                outd[s] = None
            ind[s] = fire_in(t, s)
        if t == 0:
            pltpu.sync_copy(idx_hbm.at[pl.ds(base, _B_PER_W)], idx_v)

    for s in range(_NSLOTS):
        if outd[s] is not None:
            for d in outd[s]:
                d.wait()

    for h in range(2):
        @pl.when(sid == h)
        def _(h=h):
            pltpu.sync_copy(tail_hbm.at[pl.ds(h * 8, 8), :], slots[0])
            tds = [
                pltpu.async_copy(
                    slots[0].at[r],
                    table_sh.at[pl.ds((h * 8 + r) * _PITCH + _TAIL_START,
                                      128)],
                    sem_out)
                for r in range(8)
            ]
            for d in tds:
                d.wait()

    scope_stage.__exit__(None, None, None)
    with jax.named_scope("sync"):
        plsc.subcore_barrier()

    scope_g = jax.named_scope("gatherout")
    scope_g.__enter__()
    outw = [None, None]
    for w in range(4):
        buf = words2[w % 2]
        if outw[w % 2] is not None:
            for d in outw[w % 2]:
                d.wait()
        gs = [
            pltpu.async_copy(
                table_sh.at[pl.ds((4 * w + k) * _PITCH, _PITCH)].at[idx_v],
                buf.at[pl.ds(k * _B_PER_W, _B_PER_W)],
                sem)
            for k in range(4)
        ]
        for g in gs:
            g.wait()
        outw[w % 2] = [
            pltpu.async_copy(
                buf.at[pl.ds(k * _B_PER_W, _B_PER_W)],
                outT_hbm.at[4 * w + k, pl.ds(base, _B_PER_W)],
                sem_out)
            for k in range(4)
        ]
    for ds_ in outw:
        for d in ds_:
            d.wait()
    scope_g.__exit__(None, None, None)


def kernel(annotator, z_vecs):
    zt3 = z_vecs.T.reshape(2, 8, NUM_ANNOTATORS)
    tail = jnp.pad(z_vecs.T[:, _TAIL_START:],
                   ((0, 0), (0, _PITCH - NUM_ANNOTATORS)))
    outT = _gather_sc(annotator.astype(jnp.int32), zt3, tail)
    return outT.T


# speedup vs baseline: 2.7024x; 1.4436x over previous
import functools

import jax
import jax.numpy as jnp
from jax import lax
from jax.experimental import pallas as pl
from jax.experimental.pallas import tpu as pltpu
from jax.experimental.pallas import tpu_sc as plsc

NUM_ANNOTATORS = 100000
LATENT_DIMS = 16
BATCH = 16384

_NUM_CORES = 2
_NUM_SUBCORES = 16
_NW = _NUM_CORES * _NUM_SUBCORES
_B_PER_W = BATCH // _NW
_FULL_BLOCKS = NUM_ANNOTATORS // 128
_TAIL_START = _FULL_BLOCKS * 128
_PITCH = _TAIL_START + 128
_BLK_PER_SUB = 98
_RING = 24
_DELAY = 16


@functools.partial(
    pl.kernel,
    mesh=plsc.VectorSubcoreMesh(core_axis_name="c", subcore_axis_name="s"),
    out_type=jax.ShapeDtypeStruct((LATENT_DIMS, BATCH), jnp.float32),
    scratch_types=[
        pltpu.VMEM((_B_PER_W,), jnp.int32),
        [pltpu.VMEM((4 * _B_PER_W,), jnp.float32) for _ in range(2)],
        pltpu.VMEM((_RING * 8, 128), jnp.float32),
        pltpu.VMEM_SHARED((LATENT_DIMS * _PITCH,), jnp.float32),
        pltpu.SemaphoreType.DMA,
        pltpu.SemaphoreType.DMA,
        pltpu.SemaphoreType.DMA,
    ],
)
def _gather_sc(idx_hbm, zt3_hbm, tail_hbm, outT_hbm, idx_v, words2,
               bounce_v, table_sh, sem_in, sem_out, sem):
    sid = lax.axis_index("s")
    wid = sid * _NUM_CORES + lax.axis_index("c")
    base = wid * _B_PER_W
    jb = lax.bitwise_and(sid, jnp.int32(1))
    s8 = lax.shift_right_logical(sid, 1)
    flat_jb = jb * (8 * _PITCH)

    pltpu.sync_copy(idx_hbm.at[pl.ds(base, _B_PER_W)], idx_v)

    def blk(t):
        return lax.min(s8 * _BLK_PER_SUB + t, jnp.int32(_FULL_BLOCKS - 1))

    def stage(t, _):
        @pl.when(t >= _DELAY)
        def _():
            u = t - _DELAY
            ku = lax.rem(u, jnp.int32(_RING))
            pltpu.make_async_copy(zt3_hbm.at[0, :, pl.ds(0, 128)],
                                  bounce_v.at[pl.ds(0, 8), :],
                                  sem_in).wait()
            b128 = blk(u) * 128
            for r in range(8):
                pltpu.async_copy(
                    bounce_v.at[ku * 8 + r],
                    table_sh.at[pl.ds(flat_jb + r * _PITCH + b128, 128)],
                    sem_out)

        @pl.when(t < _BLK_PER_SUB)
        def _():
            k = lax.rem(t, jnp.int32(_RING))

            @pl.when(t >= _RING)
            def _():
                pltpu.make_async_copy(zt3_hbm.at[0, :, pl.ds(0, 128)],
                                      bounce_v.at[pl.ds(0, 8), :],
                                      sem_out).wait()

            pltpu.async_copy(zt3_hbm.at[jb, :, pl.ds(blk(t) * 128, 128)],
                             bounce_v.at[pl.ds(k * 8, 8), :], sem_in)
        return 0

    lax.fori_loop(0, _BLK_PER_SUB + _DELAY, stage, 0)

    for _ in range(_RING):
        pltpu.make_async_copy(zt3_hbm.at[0, :, pl.ds(0, 128)],
                              bounce_v.at[pl.ds(0, 8), :], sem_out).wait()

    for h in range(2):
        @pl.when(sid == h)
        def _(h=h):
            pltpu.sync_copy(tail_hbm.at[pl.ds(h * 8, 8), :],
                            bounce_v.at[pl.ds(0, 8), :])
            tds = [
                pltpu.async_copy(
                    bounce_v.at[r],
                    table_sh.at[pl.ds((h * 8 + r) * _PITCH + _TAIL_START,
                                      128)],
                    sem_out)
                for r in range(8)
            ]
            for d in tds:
                d.wait()

    plsc.subcore_barrier()

    outw = [None, None]
    for w in range(4):
        buf = words2[w % 2]
        if outw[w % 2] is not None:
            for d in outw[w % 2]:
                d.wait()
        gs = [
            pltpu.async_copy(
                table_sh.at[pl.ds((4 * w + k) * _PITCH, _PITCH)].at[idx_v],
                buf.at[pl.ds(k * _B_PER_W, _B_PER_W)],
                sem)
            for k in range(4)
        ]
        for g in gs:
            g.wait()
        outw[w % 2] = [
            pltpu.async_copy(
                buf.at[pl.ds(k * _B_PER_W, _B_PER_W)],
                outT_hbm.at[4 * w + k, pl.ds(base, _B_PER_W)],
                sem_out)
            for k in range(4)
        ]
    for ds_ in outw:
        for d in ds_:
            d.wait()


def kernel(annotator, z_vecs):
    zt3 = z_vecs.T.reshape(2, 8, NUM_ANNOTATORS)
    tail = jnp.pad(z_vecs.T[:, _TAIL_START:],
                   ((0, 0), (0, _PITCH - NUM_ANNOTATORS)))
    outT = _gather_sc(annotator.astype(jnp.int32), zt3, tail)
    return outT.T


# speedup vs baseline: 2.7434x; 1.0152x over previous
import functools

import jax
import jax.numpy as jnp
from jax import lax
from jax.experimental import pallas as pl
from jax.experimental.pallas import tpu as pltpu
from jax.experimental.pallas import tpu_sc as plsc

NUM_ANNOTATORS = 100000
LATENT_DIMS = 16
BATCH = 16384

_NUM_CORES = 2
_NUM_SUBCORES = 16
_NW = _NUM_CORES * _NUM_SUBCORES
_B_PER_W = BATCH // _NW
_FULL_BLOCKS = NUM_ANNOTATORS // 128
_TAIL_START = _FULL_BLOCKS * 128
_PITCH = _TAIL_START + 128
_BLK_PER_SUB = 98
_RING = 24
_DELAY = 20


@functools.partial(
    pl.kernel,
    mesh=plsc.VectorSubcoreMesh(core_axis_name="c", subcore_axis_name="s"),
    out_type=jax.ShapeDtypeStruct((LATENT_DIMS, BATCH), jnp.float32),
    scratch_types=[
        pltpu.VMEM((_B_PER_W,), jnp.int32),
        [pltpu.VMEM((4 * _B_PER_W,), jnp.float32) for _ in range(2)],
        pltpu.VMEM((_RING * 8, 128), jnp.float32),
        pltpu.VMEM_SHARED((LATENT_DIMS * _PITCH,), jnp.float32),
        pltpu.SemaphoreType.DMA,
        pltpu.SemaphoreType.DMA,
        pltpu.SemaphoreType.DMA,
    ],
)
def _gather_sc(idx_hbm, zt3_hbm, tail_hbm, outT_hbm, idx_v, words2,
               bounce_v, table_sh, sem_in, sem_out, sem):
    sid = lax.axis_index("s")
    wid = sid * _NUM_CORES + lax.axis_index("c")
    base = wid * _B_PER_W
    jb = lax.bitwise_and(sid, jnp.int32(1))
    s8 = lax.shift_right_logical(sid, 1)
    flat_jb = jb * (8 * _PITCH)

    pltpu.sync_copy(idx_hbm.at[pl.ds(base, _B_PER_W)], idx_v)

    def blk(t):
        return lax.min(s8 * _BLK_PER_SUB + t, jnp.int32(_FULL_BLOCKS - 1))

    def stage(t, _):
        @pl.when(t >= _DELAY)
        def _():
            u = t - _DELAY
            ku = lax.rem(u, jnp.int32(_RING))
            pltpu.make_async_copy(zt3_hbm.at[0, :, pl.ds(0, 128)],
                                  bounce_v.at[pl.ds(0, 8), :],
                                  sem_in).wait()
            b128 = blk(u) * 128
            for r in range(8):
                pltpu.async_copy(
                    bounce_v.at[ku * 8 + r],
                    table_sh.at[pl.ds(flat_jb + r * _PITCH + b128, 128)],
                    sem_out)

        @pl.when(t < _BLK_PER_SUB)
        def _():
            k = lax.rem(t, jnp.int32(_RING))

            @pl.when(t >= _RING)
            def _():
                pltpu.make_async_copy(zt3_hbm.at[0, :, pl.ds(0, 128)],
                                      bounce_v.at[pl.ds(0, 8), :],
                                      sem_out).wait()

            pltpu.async_copy(zt3_hbm.at[jb, :, pl.ds(blk(t) * 128, 128)],
                             bounce_v.at[pl.ds(k * 8, 8), :], sem_in)
        return 0

    lax.fori_loop(0, _BLK_PER_SUB + _DELAY, stage, 0)

    for _ in range(_RING):
        pltpu.make_async_copy(zt3_hbm.at[0, :, pl.ds(0, 128)],
                              bounce_v.at[pl.ds(0, 8), :], sem_out).wait()

    for h in range(2):
        @pl.when(sid == h)
        def _(h=h):
            pltpu.sync_copy(tail_hbm.at[pl.ds(h * 8, 8), :],
                            bounce_v.at[pl.ds(0, 8), :])
            tds = [
                pltpu.async_copy(
                    bounce_v.at[r],
                    table_sh.at[pl.ds((h * 8 + r) * _PITCH + _TAIL_START,
                                      128)],
                    sem_out)
                for r in range(8)
            ]
            for d in tds:
                d.wait()

    plsc.subcore_barrier()

    outw = [None, None]
    for w in range(4):
        buf = words2[w % 2]
        if outw[w % 2] is not None:
            for d in outw[w % 2]:
                d.wait()
        gs = [
            pltpu.async_copy(
                table_sh.at[pl.ds((4 * w + k) * _PITCH, _PITCH)].at[idx_v],
                buf.at[pl.ds(k * _B_PER_W, _B_PER_W)],
                sem)
            for k in range(4)
        ]
        for g in gs:
            g.wait()
        outw[w % 2] = [
            pltpu.async_copy(
                buf.at[pl.ds(k * _B_PER_W, _B_PER_W)],
                outT_hbm.at[4 * w + k, pl.ds(base, _B_PER_W)],
                sem_out)
            for k in range(4)
        ]
    for ds_ in outw:
        for d in ds_:
            d.wait()


def kernel(annotator, z_vecs):
    zt3 = z_vecs.T.reshape(2, 8, NUM_ANNOTATORS)
    tail = jnp.pad(z_vecs.T[:, _TAIL_START:],
                   ((0, 0), (0, _PITCH - NUM_ANNOTATORS)))
    outT = _gather_sc(annotator.astype(jnp.int32), zt3, tail)
    return outT.T
